# baseline (speedup 1.0000x reference)
import functools

import jax
import jax.numpy as jnp
from jax import lax
from jax.experimental import pallas as pl
from jax.experimental.pallas import tpu as pltpu
from jax.experimental.pallas import tpu_sc as plsc

_NEG_INF = float("-inf")

_NC = 2
_NS = 16
_CHUNK = 3200


def _matvec_body(x_ref, w_ref, o_ref):
    o_ref[...] = jnp.dot(x_ref[...], w_ref[...].T,
                         preferred_element_type=jnp.float32)


def _series_matvec(loss_series, W_series, npad):
    n, w = loss_series.shape
    br = 4000 if n % 4000 == 0 else n
    out = pl.pallas_call(
        _matvec_body,
        grid=(n // br,),
        in_specs=[
            pl.BlockSpec((br, w), lambda i: (i, 0)),
            pl.BlockSpec((1, w), lambda i: (0, 0)),
        ],
        out_specs=pl.BlockSpec((br, 1), lambda i: (i, 0)),
        out_shape=jax.ShapeDtypeStruct((npad, 1), jnp.float32),
    )(loss_series, W_series)
    return out.reshape(-1)


def _combine_body(n, h_ref, pre_ref, suc_ref, wg_ref, o_ref):
    h = h_ref[...]
    pre = pre_ref[...]
    suc = suc_ref[...]
    r = h.shape[0]
    gid = (jax.lax.broadcasted_iota(jnp.int32, (r, 128), 0) * 128
           + jax.lax.broadcasted_iota(jnp.int32, (r, 128), 1))
    valid = gid < n
    pre = jnp.where(pre == _NEG_INF, 0.0, pre)
    suc = jnp.where(suc == _NEG_INF, 0.0, suc)
    x = h + wg_ref[0, 0] * pre + wg_ref[0, 1] * suc
    x = jnp.where(valid, x, _NEG_INF)
    m = jnp.max(x)
    e = jnp.exp(x - m)
    o_ref[...] = e / jnp.sum(e)


def _combine_softmax(n, h, pre, suc, W_graph):
    r = h.shape[0] // 128
    out = pl.pallas_call(
        functools.partial(_combine_body, n),
        in_specs=[
            pl.BlockSpec((r, 128), lambda: (0, 0)),
            pl.BlockSpec((r, 128), lambda: (0, 0)),
            pl.BlockSpec((r, 128), lambda: (0, 0)),
            pl.BlockSpec(memory_space=pltpu.SMEM),
        ],
        out_specs=pl.BlockSpec((r, 128), lambda: (0, 0)),
        out_shape=jax.ShapeDtypeStruct((r, 128), jnp.float32),
    )(h.reshape(r, 128), pre.reshape(r, 128), suc.reshape(r, 128), W_graph)
    return out.reshape(-1)[:n]


def _sc_scatter_max(feat, ei_flat):
    npad = feat.shape[0]
    e = ei_flat.shape[0] // 2
    sl = npad // _NS
    assert sl % 8 == 0
    ept = e // _NS
    assert ept * _NS == e and ept % _CHUNK == 0
    nchunk = ept // _CHUNK
    nvec = _CHUNK // 16
    q = npad // 4
    assert q * 4 == npad and q % 8 == 0

    assert nchunk >= 3
    G = 2
    assert nvec % G == 0
    rh0 = ((sl // 2 + 15) // 16) * 16
    rh0 = min(rh0, _CHUNK)
    rh1 = sl - rh0
    assert rh0 % 16 == 0 and rh1 % 16 == 0 and rh0 % 8 == 0
    assert 0 < rh1 <= _CHUNK

    def body(feat_hbm, ei_hbm, pre_hbm, suc_hbm, parts_hbm,
             feat_s, acc,
             srcb0, srcb1, dstb0, dstb1, vals0, vals1,
             slin0, slin1, sind0, sind1):
        srcbs = (srcb0, srcb1)
        dstbs = (dstb0, dstb1)
        valss = (vals0, vals1)
        slins = (slin0, slin1)
        sinds = (sind0, sind1)
        c = lax.axis_index("c")
        s = lax.axis_index("s")
        is_pre = c == 0
        base0 = s * ept
        ncl = nchunk - 1

        def issue_lin(k, si):
            base = base0 + jnp.minimum(k, ncl) * _CHUNK
            @pl.when(is_pre)
            def _():
                pltpu.async_copy(ei_hbm.at[pl.ds(base, _CHUNK)], srcbs[si],
                                 slins[si])
                pltpu.async_copy(ei_hbm.at[pl.ds(e + base, _CHUNK)],
                                 dstbs[si], slins[si])
            @pl.when(jnp.logical_not(is_pre))
            def _():
                pltpu.async_copy(ei_hbm.at[pl.ds(e + base, _CHUNK)],
                                 srcbs[si], slins[si])
                pltpu.async_copy(ei_hbm.at[pl.ds(base, _CHUNK)], dstbs[si],
                                 slins[si])

        def wait_lin(si):
            pltpu.make_async_copy(ei_hbm.at[pl.ds(0, _CHUNK)], srcbs[si],
                                  slins[si]).wait()
            pltpu.make_async_copy(ei_hbm.at[pl.ds(0, _CHUNK)], dstbs[si],
                                  slins[si]).wait()

        issue_lin(0, 0)
        issue_lin(1, 1)

        @pl.when(s < 4)
        def _():
            pltpu.sync_copy(feat_hbm.at[pl.ds(s * q, q)], acc.at[pl.ds(0, q)])
            pltpu.sync_copy(acc.at[pl.ds(0, q)], feat_s.at[pl.ds(s * q, q)])

        neg = jnp.full((16,), _NEG_INF, jnp.float32)
        assert npad % (16 * 8) == 0
        def init_j(j, carry):
            o = j * 128
            for u in range(8):
                acc[pl.ds(o + u * 16, 16)] = neg
            return carry
        lax.fori_loop(0, npad // 128, init_j, 0)

        plsc.subcore_barrier()

        def fire_ind(si):
            for m in range(_CHUNK // 128):
                pltpu.async_copy(
                    feat_s.at[srcbs[si].at[pl.ds(m * 128, 128)]],
                    valss[si].at[pl.ds(m * 128, 128)], sinds[si])

        def drain_ind(si):
            pltpu.make_async_copy(feat_hbm.at[pl.ds(0, _CHUNK)], valss[si],
                                  sinds[si]).wait()

        def compute(si):
            srcb, dstb, vals = srcbs[si], dstbs[si], valss[si]

            def step(j, bad):
                offb = j * (16 * G)
                keys, masks, vls, wins = [], [], [], []
                for g in range(G):
                    o = offb + g * 16
                    sv = srcb[pl.ds(o, 16)]
                    key = dstb[pl.ds(o, 16)]
                    vl = vals[pl.ds(o, 16)]
                    mask = sv != key
                    cur = plsc.load_gather(acc, [key])
                    keys.append(key)
                    masks.append(mask)
                    vls.append(vl)
                    wins.append(jnp.logical_and(mask, vl > cur))
                for g in range(G):
                    plsc.store_scatter(acc, [keys[g]], vls[g], mask=wins[g])
                for g in range(G):
                    cur2 = plsc.load_gather(acc, [keys[g]])
                    bad = bad + jnp.where(
                        jnp.logical_and(masks[g], vls[g] > cur2), 1, 0)
                return bad

            badv = lax.fori_loop(0, nvec // G, step,
                                 jnp.zeros((16,), jnp.int32))

            def fix(t):
                bv = lax.fori_loop(0, nvec // G, step,
                                   jnp.zeros((16,), jnp.int32))
                return jnp.max(bv)

            lax.while_loop(lambda t: t > 0, fix, jnp.max(badv))

        wait_lin(0)
        fire_ind(0)

        def substep(k, si):
            sj = (si + 1) % 2
            drain_ind(si)
            wait_lin(sj)
            fire_ind(sj)
            compute(si)
            issue_lin(k + 2, si)

        def group_body(gidx, carry):
            k = gidx * 2
            substep(k, 0)
            substep(k + 1, 1)
            return carry

        ngroup = nchunk // 2
        lax.fori_loop(0, ngroup, group_body, 0)
        for k in range(ngroup * 2, nchunk):
            substep(k, k % 2)

        lin_iss = [0, 0]
        lin_wt = [0, 0]
        ind_iss = [0, 0]
        ind_wt = [0, 0]
        lin_iss[0] += 1; lin_iss[1] += 1
        lin_wt[0] += 1
        ind_iss[0] += 1
        for k in range(nchunk):
            si = k % 2
            sj = (si + 1) % 2
            ind_wt[si] += 1
            lin_wt[sj] += 1
            ind_iss[sj] += 1
            lin_iss[si] += 1
        for si in range(2):
            for _ in range(ind_iss[si] - ind_wt[si]):
                drain_ind(si)
            for _ in range(lin_iss[si] - lin_wt[si]):
                wait_lin(si)

        pbase = (c * _NS + s) * npad
        pltpu.sync_copy(acc, parts_hbm.at[pl.ds(pbase, npad)])
        plsc.subcore_barrier()

        off = s * sl
        cbase = c * _NS * npad
        pltpu.sync_copy(parts_hbm.at[pl.ds(cbase + off, sl)],
                        acc.at[pl.ds(0, sl)])
        seq = [(t, h) for t in range(1, _NS) for h in range(2)]
        def rseg(i):
            t, h = seq[i]
            roff = cbase + t * npad + off + h * rh0
            rlen = rh0 if h == 0 else rh1
            return roff, rlen, valss[i % 2], slins[i % 2]
        def rissue(i):
            roff, rlen, dstv, sem = rseg(i)
            pltpu.async_copy(parts_hbm.at[pl.ds(roff, rlen)],
                             dstv.at[pl.ds(0, rlen)], sem)
        def rwait(i):
            roff, rlen, dstv, sem = rseg(i)
            pltpu.make_async_copy(parts_hbm.at[pl.ds(roff, rlen)],
                                  dstv.at[pl.ds(0, rlen)], sem).wait()
        for i in range(min(2, len(seq))):
            rissue(i)
        for i, (t, h) in enumerate(seq):
            rwait(i)
            _, rlen, dstv, _sem = rseg(i)
            abase = 0 if h == 0 else rh0
            def red_j(j, carry, _dstv=dstv, _abase=abase):
                o = j * 16
                acc[pl.ds(_abase + o, 16)] = jnp.maximum(
                    acc[pl.ds(_abase + o, 16)], _dstv[pl.ds(o, 16)])
                return carry
            lax.fori_loop(0, rlen // 16, red_j, 0)
            if i + 2 < len(seq):
                rissue(i + 2)

        @pl.when(is_pre)
        def _():
            pltpu.sync_copy(acc.at[pl.ds(0, sl)], pre_hbm.at[pl.ds(off, sl)])
        @pl.when(jnp.logical_not(is_pre))
        def _():
            pltpu.sync_copy(acc.at[pl.ds(0, sl)], suc_hbm.at[pl.ds(off, sl)])

    mesh = plsc.VectorSubcoreMesh(core_axis_name="c", subcore_axis_name="s",
                                  num_cores=_NC, num_subcores=_NS)
    pre, suc, _ = pl.kernel(
        body,
        out_type=[jax.ShapeDtypeStruct((npad,), jnp.float32),
                  jax.ShapeDtypeStruct((npad,), jnp.float32),
                  jax.ShapeDtypeStruct((_NC * _NS * npad,), jnp.float32)],
        mesh=mesh,
        compiler_params=pltpu.CompilerParams(needs_layout_passes=False),
        scratch_types=(
            [pltpu.VMEM_SHARED((npad,), jnp.float32),
             pltpu.VMEM((npad,), jnp.float32)]
            + [pltpu.VMEM((_CHUNK,), jnp.int32)] * 4
            + [pltpu.VMEM((_CHUNK,), jnp.float32)] * 2
            + [pltpu.SemaphoreType.DMA] * 4
        ),
    )(feat, ei_flat)
    return pre, suc


def kernel(loss_series, edge_index, W_series, W_graph):
    n = loss_series.shape[0]
    npad = ((n + _NS * 16 - 1) // (_NS * 16)) * (_NS * 16)
    feat = _series_matvec(loss_series, W_series, npad)
    ei_flat = edge_index.reshape(-1)
    pre_p, suc_p = _sc_scatter_max(feat, ei_flat)
    out = _combine_softmax(n, feat, pre_p, suc_p, W_graph)
    return out[:, None]


# speedup vs baseline: 1.2474x; 1.2474x over previous
import functools

import jax
import jax.numpy as jnp
from jax import lax
from jax.experimental import pallas as pl
from jax.experimental.pallas import tpu as pltpu
from jax.experimental.pallas import tpu_sc as plsc

_NEG_INF = float("-inf")

_NC = 2
_NS = 16
_CHUNK = 3200


def _matvec_body(x_ref, w_ref, o_ref):
    r = jax.lax.dot_general(
        w_ref[...], x_ref[...], (((1,), (1,)), ((), ())),
        preferred_element_type=jnp.float32)
    o_ref[...] = r[None]


def _series_matvec(loss_series, W_series):
    n, w = loss_series.shape
    br = 4000 if n % 4000 == 0 else n
    out = pl.pallas_call(
        _matvec_body,
        grid=(n // br,),
        in_specs=[
            pl.BlockSpec((br, w), lambda i: (i, 0)),
            pl.BlockSpec((1, w), lambda i: (0, 0)),
        ],
        out_specs=pl.BlockSpec((1, 1, br), lambda i: (i, 0, 0)),
        out_shape=jax.ShapeDtypeStruct((n // br, 1, br), jnp.float32),
    )(loss_series, W_series)
    return out.reshape(-1)


def _combine_body(n, h_ref, pre_ref, suc_ref, wg_ref, o_ref):
    h = h_ref[...]
    pre = pre_ref[...]
    suc = suc_ref[...]
    r = h.shape[0]
    gid = (jax.lax.broadcasted_iota(jnp.int32, (r, 128), 0) * 128
           + jax.lax.broadcasted_iota(jnp.int32, (r, 128), 1))
    valid = gid < n
    pre = jnp.where(pre == _NEG_INF, 0.0, pre)
    suc = jnp.where(suc == _NEG_INF, 0.0, suc)
    x = h + wg_ref[0, 0] * pre + wg_ref[0, 1] * suc
    x = jnp.where(valid, x, _NEG_INF)
    m = jnp.max(x)
    e = jnp.exp(x - m)
    o_ref[...] = e / jnp.sum(e)


def _combine_softmax(n, h, pre, suc, W_graph):
    r = h.shape[0] // 128
    out = pl.pallas_call(
        functools.partial(_combine_body, n),
        in_specs=[
            pl.BlockSpec((r, 128), lambda: (0, 0)),
            pl.BlockSpec((r, 128), lambda: (0, 0)),
            pl.BlockSpec((r, 128), lambda: (0, 0)),
            pl.BlockSpec(memory_space=pltpu.SMEM),
        ],
        out_specs=pl.BlockSpec((r, 128), lambda: (0, 0)),
        out_shape=jax.ShapeDtypeStruct((r, 128), jnp.float32),
    )(h.reshape(r, 128), pre.reshape(r, 128), suc.reshape(r, 128), W_graph)
    return out.reshape(-1)[:n]


def _sc_scatter_max(feat, ei_flat):
    n = feat.shape[0]
    npad = ((n + _NS * 16 - 1) // (_NS * 16)) * (_NS * 16)
    e = ei_flat.shape[0] // 2
    sl = npad // _NS
    assert sl % 8 == 0
    ept = e // _NS
    assert ept * _NS == e and ept % _CHUNK == 0
    nchunk = ept // _CHUNK
    nvec = _CHUNK // 16
    q = n // 4
    assert q * 4 == n and q % 8 == 0

    assert nchunk >= 3
    G = 4
    assert nvec % G == 0
    rh0 = ((sl // 2 + 15) // 16) * 16
    rh0 = min(rh0, _CHUNK)
    rh1 = sl - rh0
    assert rh0 % 16 == 0 and rh1 % 16 == 0 and rh0 % 8 == 0
    assert 0 < rh1 <= _CHUNK

    def body(feat_hbm, ei_hbm, pre_hbm, suc_hbm, parts_hbm,
             feat_s, acc,
             srcb0, srcb1, dstb0, dstb1, vals0, vals1,
             slin0, slin1, sind0, sind1):
        srcbs = (srcb0, srcb1)
        dstbs = (dstb0, dstb1)
        valss = (vals0, vals1)
        slins = (slin0, slin1)
        sinds = (sind0, sind1)
        c = lax.axis_index("c")
        s = lax.axis_index("s")
        is_pre = c == 0
        base0 = s * ept
        ncl = nchunk - 1

        def issue_lin(k, si):
            base = base0 + jnp.minimum(k, ncl) * _CHUNK
            @pl.when(is_pre)
            def _():
                pltpu.async_copy(ei_hbm.at[pl.ds(base, _CHUNK)], srcbs[si],
                                 slins[si])
                pltpu.async_copy(ei_hbm.at[pl.ds(e + base, _CHUNK)],
                                 dstbs[si], slins[si])
            @pl.when(jnp.logical_not(is_pre))
            def _():
                pltpu.async_copy(ei_hbm.at[pl.ds(e + base, _CHUNK)],
                                 srcbs[si], slins[si])
                pltpu.async_copy(ei_hbm.at[pl.ds(base, _CHUNK)], dstbs[si],
                                 slins[si])

        def wait_lin(si):
            pltpu.make_async_copy(ei_hbm.at[pl.ds(0, _CHUNK)], srcbs[si],
                                  slins[si]).wait()
            pltpu.make_async_copy(ei_hbm.at[pl.ds(0, _CHUNK)], dstbs[si],
                                  slins[si]).wait()

        issue_lin(0, 0)
        issue_lin(1, 1)

        @pl.when(s < 4)
        def _():
            pltpu.sync_copy(feat_hbm.at[pl.ds(s * q, q)], acc.at[pl.ds(0, q)])
            pltpu.sync_copy(acc.at[pl.ds(0, q)], feat_s.at[pl.ds(s * q, q)])

        neg = jnp.full((16,), _NEG_INF, jnp.float32)
        assert npad % (16 * 8) == 0
        def init_j(j, carry):
            o = j * 128
            for u in range(8):
                acc[pl.ds(o + u * 16, 16)] = neg
            return carry
        lax.fori_loop(0, npad // 128, init_j, 0)

        plsc.subcore_barrier()

        def fire_ind(si):
            for m in range(_CHUNK // 128):
                pltpu.async_copy(
                    feat_s.at[srcbs[si].at[pl.ds(m * 128, 128)]],
                    valss[si].at[pl.ds(m * 128, 128)], sinds[si])

        def drain_ind(si):
            pltpu.make_async_copy(feat_hbm.at[pl.ds(0, _CHUNK)], valss[si],
                                  sinds[si]).wait()

        def compute(si):
            srcb, dstb, vals = srcbs[si], dstbs[si], valss[si]

            def step(j, bad):
                offb = j * (16 * G)
                keys, masks, vls, wins = [], [], [], []
                for g in range(G):
                    o = offb + g * 16
                    sv = srcb[pl.ds(o, 16)]
                    key = dstb[pl.ds(o, 16)]
                    vl = vals[pl.ds(o, 16)]
                    mask = sv != key
                    cur = plsc.load_gather(acc, [key])
                    keys.append(key)
                    masks.append(mask)
                    vls.append(vl)
                    wins.append(jnp.logical_and(mask, vl > cur))
                for g in range(G):
                    plsc.store_scatter(acc, [keys[g]], vls[g], mask=wins[g])
                for g in range(G):
                    cur2 = plsc.load_gather(acc, [keys[g]])
                    bad = bad + jnp.where(
                        jnp.logical_and(masks[g], vls[g] > cur2), 1, 0)
                return bad

            badv = lax.fori_loop(0, nvec // G, step,
                                 jnp.zeros((16,), jnp.int32))

            def fix(t):
                bv = lax.fori_loop(0, nvec // G, step,
                                   jnp.zeros((16,), jnp.int32))
                return jnp.max(bv)

            lax.while_loop(lambda t: t > 0, fix, jnp.max(badv))

        wait_lin(0)
        fire_ind(0)

        def substep(k, si):
            sj = (si + 1) % 2
            drain_ind(si)
            wait_lin(sj)
            fire_ind(sj)
            compute(si)
            issue_lin(k + 2, si)

        def group_body(gidx, carry):
            k = gidx * 2
            substep(k, 0)
            substep(k + 1, 1)
            return carry

        ngroup = nchunk // 2
        lax.fori_loop(0, ngroup, group_body, 0)
        for k in range(ngroup * 2, nchunk):
            substep(k, k % 2)

        lin_iss = [0, 0]
        lin_wt = [0, 0]
        ind_iss = [0, 0]
        ind_wt = [0, 0]
        lin_iss[0] += 1; lin_iss[1] += 1
        lin_wt[0] += 1
        ind_iss[0] += 1
        for k in range(nchunk):
            si = k % 2
            sj = (si + 1) % 2
            ind_wt[si] += 1
            lin_wt[sj] += 1
            ind_iss[sj] += 1
            lin_iss[si] += 1
        for si in range(2):
            for _ in range(ind_iss[si] - ind_wt[si]):
                drain_ind(si)
            for _ in range(lin_iss[si] - lin_wt[si]):
                wait_lin(si)

        pbase = (c * _NS + s) * npad
        pltpu.sync_copy(acc, parts_hbm.at[pl.ds(pbase, npad)])
        plsc.subcore_barrier()

        off = s * sl
        cbase = c * _NS * npad
        pltpu.sync_copy(parts_hbm.at[pl.ds(cbase + off, sl)],
                        acc.at[pl.ds(0, sl)])
        seq = [(t, h) for t in range(1, _NS) for h in range(2)]
        def rseg(i):
            t, h = seq[i]
            roff = cbase + t * npad + off + h * rh0
            rlen = rh0 if h == 0 else rh1
            return roff, rlen, valss[i % 2], slins[i % 2]
        def rissue(i):
            roff, rlen, dstv, sem = rseg(i)
            pltpu.async_copy(parts_hbm.at[pl.ds(roff, rlen)],
                             dstv.at[pl.ds(0, rlen)], sem)
        def rwait(i):
            roff, rlen, dstv, sem = rseg(i)
            pltpu.make_async_copy(parts_hbm.at[pl.ds(roff, rlen)],
                                  dstv.at[pl.ds(0, rlen)], sem).wait()
        for i in range(min(2, len(seq))):
            rissue(i)
        for i, (t, h) in enumerate(seq):
            rwait(i)
            _, rlen, dstv, _sem = rseg(i)
            abase = 0 if h == 0 else rh0
            def red_j(j, carry, _dstv=dstv, _abase=abase):
                o = j * 16
                acc[pl.ds(_abase + o, 16)] = jnp.maximum(
                    acc[pl.ds(_abase + o, 16)], _dstv[pl.ds(o, 16)])
                return carry
            lax.fori_loop(0, rlen // 16, red_j, 0)
            if i + 2 < len(seq):
                rissue(i + 2)

        @pl.when(is_pre)
        def _():
            pltpu.sync_copy(acc.at[pl.ds(0, sl)], pre_hbm.at[pl.ds(off, sl)])
        @pl.when(jnp.logical_not(is_pre))
        def _():
            pltpu.sync_copy(acc.at[pl.ds(0, sl)], suc_hbm.at[pl.ds(off, sl)])

    mesh = plsc.VectorSubcoreMesh(core_axis_name="c", subcore_axis_name="s",
                                  num_cores=_NC, num_subcores=_NS)
    pre, suc, _ = pl.kernel(
        body,
        out_type=[jax.ShapeDtypeStruct((npad,), jnp.float32),
                  jax.ShapeDtypeStruct((npad,), jnp.float32),
                  jax.ShapeDtypeStruct((_NC * _NS * npad,), jnp.float32)],
        mesh=mesh,
        compiler_params=pltpu.CompilerParams(needs_layout_passes=False),
        scratch_types=(
            [pltpu.VMEM_SHARED((n,), jnp.float32),
             pltpu.VMEM((npad,), jnp.float32)]
            + [pltpu.VMEM((_CHUNK,), jnp.int32)] * 4
            + [pltpu.VMEM((_CHUNK,), jnp.float32)] * 2
            + [pltpu.SemaphoreType.DMA] * 4
        ),
    )(feat, ei_flat)
    return pre, suc


def kernel(loss_series, edge_index, W_series, W_graph):
    n = loss_series.shape[0]
    npad = ((n + _NS * 16 - 1) // (_NS * 16)) * (_NS * 16)
    feat = _series_matvec(loss_series, W_series)
    ei_flat = edge_index.reshape(-1)
    pre_p, suc_p = _sc_scatter_max(feat, ei_flat)
    hp = jnp.pad(feat, (0, npad - n))
    out = _combine_softmax(n, hp, pre_p, suc_p, W_graph)
    return out[:, None]
